# baseline (speedup 1.0000x reference)
import functools

import jax
import jax.numpy as jnp
from jax import lax
from jax.experimental import pallas as pl
from jax.experimental.pallas import tpu as pltpu
from jax.experimental.pallas import tpu_sc as plsc

B = 16384
L = 200
NW = 32
ROWS_PER_TILE = B // NW
CH = 64
NCHUNK = ROWS_PER_TILE // CH
G = CH // 16
UNROLL = 8


def _sc_embed_pool_linear(ids, tbl_flat, wb):
    mesh = plsc.VectorSubcoreMesh(core_axis_name="c", subcore_axis_name="s")
    out_sds = jax.ShapeDtypeStruct((B,), jnp.float32)

    @functools.partial(
        pl.kernel,
        mesh=mesh,
        out_type=(out_sds, out_sds, out_sds),
        compiler_params=pltpu.CompilerParams(needs_layout_passes=False),
        scratch_types=[
            pltpu.VMEM((CH, L), jnp.int32),
            pltpu.VMEM((CH, L), jnp.int32),
            pltpu.VMEM((2 * CH,), jnp.float32),
            pltpu.VMEM((2 * CH,), jnp.float32),
            pltpu.VMEM((2 * CH,), jnp.float32),
            pltpu.VMEM((1024,), jnp.float32),
            pltpu.VMEM((432,), jnp.float32),
            pltpu.VMEM((2048,), jnp.float32),
            pltpu.VMEM((2048,), jnp.float32),
            pltpu.VMEM((2048,), jnp.float32),
            pltpu.VMEM((G * 2048,), jnp.float32),
            pltpu.SemaphoreType.DMA,
            pltpu.SemaphoreType.DMA,
            pltpu.SemaphoreType.DMA,
            pltpu.SemaphoreType.DMA,
        ],
    )
    def body(ids_hbm, tbl_hbm, wb_hbm, o0_hbm, o1_hbm, o2_hbm,
             ids_a, ids_b, ov0, ov1, ov2, tbl_v, wb_v, m0, m1, m2, hist,
             sia, sib, soa, sob):
        wid = lax.axis_index("s") * 2 + lax.axis_index("c")
        iota = jnp.arange(16, dtype=jnp.int32)

        pltpu.sync_copy(tbl_hbm, tbl_v)
        pltpu.sync_copy(wb_hbm, wb_v)

        wvec = [[wb_v[pl.ds((d * 3 + c) * 16, 16)]
                 for c in range(3)] for d in range(8)]
        bvec = [wb_v[pl.ds((24 + c) * 16, 16)] for c in range(3)]
        m_refs = (m0, m1, m2)
        inv_l = jnp.float32(1.0 / L)
        dnums = lax.GatherDimensionNumbers(
            offset_dims=(), collapsed_slice_dims=(0,), start_index_map=(0,))

        def fold_chunk(k, _):
            vb = (iota + k * 16) * 8
            acc = [jnp.zeros((16,), jnp.float32) for _ in range(3)]
            for d in range(8):
                col = plsc.load_gather(tbl_v, [vb + d])
                for c in range(3):
                    acc[c] = acc[c] + col * wvec[d][c]
            mvs = [(acc[c] + bvec[c]) * inv_l for c in range(3)]

            def rep_one(j, _):
                jv = jnp.broadcast_to(j.astype(jnp.int32), (16, 1))
                for c in range(3):
                    bj = lax.gather(
                        mvs[c], jv, dnums, (1,),
                        mode=lax.GatherScatterMode.PROMISE_IN_BOUNDS)
                    m_refs[c][pl.ds((k * 16 + j) * 16, 16)] = bj
                return 0

            lax.fori_loop(0, 16, rep_one, 0)
            return 0

        lax.fori_loop(0, 7, fold_chunk, 0)

        zero = jnp.zeros((16,), jnp.float32)
        row0 = wid * ROWS_PER_TILE
        ids_bufs = (ids_a, ids_b)
        ids_sems = (sia, sib)
        out_sems = (soa, sob)
        o_hbms = (o0_hbm, o1_hbm, o2_hbm)
        o_vs = (ov0, ov1, ov2)

        def start_ids(ch):
            return pltpu.async_copy(
                ids_hbm.at[pl.ds(row0 + ch * CH, CH), :],
                ids_bufs[ch % 2], ids_sems[ch % 2])

        def zinit(i, _):
            hist[pl.ds(i * 16, 16)] = zero
            return 0

        lax.fori_loop(0, G * 128, zinit, 0)

        handles = {0: start_ids(0)}
        out_handles = {}
        ones = jnp.ones((16,), jnp.float32)
        c0 = iota * 5
        for ch in range(NCHUNK):
            handles[ch].wait()
            if ch + 1 < NCHUNK:
                handles[ch + 1] = start_ids(ch + 1)
            ids_v = ids_bufs[ch % 2]
            par = ch % 2

            def group(g, _, ids_v=ids_v):
                rows = g * 16 + iota
                binbase = g * 2048 + iota

                def step(i, _):
                    l0 = i * UNROLL
                    for u in range(UNROLL):
                        raw = c0 + (l0 + u)
                        col = jnp.where(raw < L, raw, raw - L)
                        ids16 = plsc.load_gather(ids_v, [rows, col])
                        plsc.addupdate_scatter(
                            hist, [ids16 * 16 + binbase], ones)
                    return 0

                lax.fori_loop(0, L // UNROLL, step, 0)
                return 0

            lax.fori_loop(0, G, group, 0)

            if ch - 2 in out_handles:
                for h in out_handles[ch - 2]:
                    h.wait()

            def mstep(v, carry):
                accs = list(carry)
                mb = [m_refs[c][pl.ds(v * 16, 16)] for c in range(3)]
                for gi in range(G):
                    hv = hist[pl.ds(gi * 2048 + v * 16, 16)]
                    hist[pl.ds(gi * 2048 + v * 16, 16)] = zero
                    for c in range(3):
                        accs[gi * 3 + c] = accs[gi * 3 + c] + hv * mb[c]
                return tuple(accs)

            accs = lax.fori_loop(0, 100, mstep, (zero,) * (G * 3))
            for gi in range(G):
                for c in range(3):
                    o_vs[c][pl.ds(par * CH + gi * 16, 16)] = accs[gi * 3 + c]
            out_handles[ch] = tuple(
                pltpu.async_copy(
                    o_vs[c].at[pl.ds(par * CH, CH)],
                    o_hbms[c].at[pl.ds(row0 + ch * CH, CH)],
                    out_sems[par])
                for c in range(3))
        for ch in (NCHUNK - 2, NCHUNK - 1):
            for h in out_handles[ch]:
                h.wait()

    return body(ids, tbl_flat, wb)


def kernel(input_ids, attention_mask, embed_table, fc_W, fc_b):
    del attention_mask
    ids = input_ids.astype(jnp.int32)
    tbl_flat = jnp.pad(embed_table.astype(jnp.float32),
                       ((0, 28), (0, 0))).reshape(-1)
    wvals = jnp.concatenate([
        fc_W.astype(jnp.float32).reshape(-1),
        fc_b.astype(jnp.float32),
    ])
    wb = jnp.broadcast_to(wvals[:, None], (27, 16)).reshape(-1)
    o0, o1, o2 = _sc_embed_pool_linear(ids, tbl_flat, wb)
    return jnp.stack([o0, o1, o2], axis=-1)


# speedup vs baseline: 1.6658x; 1.6658x over previous
import functools

import jax
import jax.numpy as jnp
from jax import lax
from jax.experimental import pallas as pl
from jax.experimental.pallas import tpu as pltpu
from jax.experimental.pallas import tpu_sc as plsc

B = 16384
L = 200
NW = 32
ROWS_PER_TILE = B // NW
CH = 64
NCHUNK = ROWS_PER_TILE // CH
G = CH // 16
UNROLL = 8


def _sc_embed_pool_linear(ids, tbl_flat, wb):
    mesh = plsc.VectorSubcoreMesh(core_axis_name="c", subcore_axis_name="s")
    out_sds = jax.ShapeDtypeStruct((B,), jnp.float32)

    @functools.partial(
        pl.kernel,
        mesh=mesh,
        out_type=(out_sds, out_sds, out_sds),
        compiler_params=pltpu.CompilerParams(needs_layout_passes=False),
        scratch_types=[
            pltpu.VMEM((CH, L), jnp.int32),
            pltpu.VMEM((CH, L), jnp.int32),
            pltpu.VMEM((2 * CH,), jnp.float32),
            pltpu.VMEM((2 * CH,), jnp.float32),
            pltpu.VMEM((2 * CH,), jnp.float32),
            pltpu.VMEM((1024,), jnp.float32),
            pltpu.VMEM((432,), jnp.float32),
            pltpu.VMEM((2048,), jnp.float32),
            pltpu.VMEM((2048,), jnp.float32),
            pltpu.VMEM((2048,), jnp.float32),
            pltpu.SemaphoreType.DMA,
            pltpu.SemaphoreType.DMA,
            pltpu.SemaphoreType.DMA,
            pltpu.SemaphoreType.DMA,
        ],
    )
    def body(ids_hbm, tbl_hbm, wb_hbm, o0_hbm, o1_hbm, o2_hbm,
             ids_a, ids_b, ov0, ov1, ov2, tbl_v, wb_v, m0, m1, m2,
             sia, sib, soa, sob):
        wid = lax.axis_index("s") * 2 + lax.axis_index("c")
        iota = jnp.arange(16, dtype=jnp.int32)

        pltpu.sync_copy(tbl_hbm, tbl_v)
        pltpu.sync_copy(wb_hbm, wb_v)

        wvec = [[wb_v[pl.ds((d * 3 + c) * 16, 16)]
                 for c in range(3)] for d in range(8)]
        bvec = [wb_v[pl.ds((24 + c) * 16, 16)] for c in range(3)]
        m_refs = (m0, m1, m2)
        inv_l = jnp.float32(1.0 / L)
        dnums = lax.GatherDimensionNumbers(
            offset_dims=(), collapsed_slice_dims=(0,), start_index_map=(0,))

        def fold_chunk(k, _):
            vb = (iota + k * 16) * 8
            acc = [jnp.zeros((16,), jnp.float32) for _ in range(3)]
            for d in range(8):
                col = plsc.load_gather(tbl_v, [vb + d])
                for c in range(3):
                    acc[c] = acc[c] + col * wvec[d][c]
            mvs = [(acc[c] + bvec[c]) * inv_l for c in range(3)]

            def rep_one(j, _):
                jv = jnp.broadcast_to(j.astype(jnp.int32), (16, 1))
                for c in range(3):
                    bj = lax.gather(
                        mvs[c], jv, dnums, (1,),
                        mode=lax.GatherScatterMode.PROMISE_IN_BOUNDS)
                    m_refs[c][pl.ds((k * 16 + j) * 16, 16)] = bj
                return 0

            lax.fori_loop(0, 16, rep_one, 0)
            return 0

        lax.fori_loop(0, 7, fold_chunk, 0)

        zero = jnp.zeros((16,), jnp.float32)
        row0 = wid * ROWS_PER_TILE
        ids_bufs = (ids_a, ids_b)
        ids_sems = (sia, sib)
        out_sems = (soa, sob)
        o_hbms = (o0_hbm, o1_hbm, o2_hbm)
        o_vs = (ov0, ov1, ov2)

        def start_ids(ch):
            return pltpu.async_copy(
                ids_hbm.at[pl.ds(row0 + ch * CH, CH), :],
                ids_bufs[ch % 2], ids_sems[ch % 2])

        handles = {0: start_ids(0)}
        out_handles = {}
        c0 = iota * 5
        for ch in range(NCHUNK):
            handles[ch].wait()
            if ch + 1 < NCHUNK:
                handles[ch + 1] = start_ids(ch + 1)
            ids_v = ids_bufs[ch % 2]
            par = ch % 2
            if ch - 2 in out_handles:
                for h in out_handles[ch - 2]:
                    h.wait()

            def group(g, _, ids_v=ids_v, par=par):
                rows = g * 16 + iota

                def step(i, carry):
                    a0, a1, a2 = carry
                    l0 = i * UNROLL
                    for u in range(UNROLL):
                        raw = c0 + (l0 + u)
                        col = jnp.where(raw < L, raw, raw - L)
                        ids16 = plsc.load_gather(ids_v, [rows, col])
                        mi = ids16 * 16 + iota
                        a0 = a0 + plsc.load_gather(m0, [mi])
                        a1 = a1 + plsc.load_gather(m1, [mi])
                        a2 = a2 + plsc.load_gather(m2, [mi])
                    return (a0, a1, a2)

                a0, a1, a2 = lax.fori_loop(0, L // UNROLL, step,
                                           (zero, zero, zero))
                for c, a in ((0, a0), (1, a1), (2, a2)):
                    o_vs[c][pl.ds(par * CH + g * 16, 16)] = a
                return 0

            lax.fori_loop(0, G, group, 0)
            out_handles[ch] = tuple(
                pltpu.async_copy(
                    o_vs[c].at[pl.ds(par * CH, CH)],
                    o_hbms[c].at[pl.ds(row0 + ch * CH, CH)],
                    out_sems[par])
                for c in range(3))
        for ch in (NCHUNK - 2, NCHUNK - 1):
            for h in out_handles[ch]:
                h.wait()

    return body(ids, tbl_flat, wb)


def kernel(input_ids, attention_mask, embed_table, fc_W, fc_b):
    del attention_mask
    ids = input_ids.astype(jnp.int32)
    tbl_flat = jnp.pad(embed_table.astype(jnp.float32),
                       ((0, 28), (0, 0))).reshape(-1)
    wvals = jnp.concatenate([
        fc_W.astype(jnp.float32).reshape(-1),
        fc_b.astype(jnp.float32),
    ])
    wb = jnp.broadcast_to(wvals[:, None], (27, 16)).reshape(-1)
    o0, o1, o2 = _sc_embed_pool_linear(ids, tbl_flat, wb)
    return jnp.stack([o0, o1, o2], axis=-1)
